# baseline (speedup 1.0000x reference)
import dataclasses

import jax
import jax.numpy as jnp
from jax import lax
from jax.experimental import pallas as pl
from jax.experimental.pallas import tpu as pltpu
from jax.experimental.pallas import tpu_sc as plsc

EPS = 1e-07
N_DATA = 100000
DIM = 128
NCE_K = 512
K1 = NCE_K + 1
KP = 528
NCE_T = 0.07
MOMENTUM = 0.5
BATCH = 1024
NW = 32
B_PER_W = BATCH // NW
LANES = 16
CHUNKS = (128, 128, 128, 128, 16)


def _proj_body(fs_ref, ft_ref, ws_ref, wt_ref, bs_ref, bt_ref, v1_ref, v2_ref):
    x1 = jnp.dot(fs_ref[...], ws_ref[...], preferred_element_type=jnp.float32)
    x1 = x1 + bs_ref[...]
    v1_ref[...] = x1 / jnp.sqrt(jnp.sum(x1 * x1, axis=1, keepdims=True))
    x2 = jnp.dot(ft_ref[...], wt_ref[...], preferred_element_type=jnp.float32)
    x2 = x2 + bt_ref[...]
    v2_ref[...] = x2 / jnp.sqrt(jnp.sum(x2 * x2, axis=1, keepdims=True))


def _project(f_s, f_t, WsT, WtT, bs, bt):
    nb = 4
    bb = BATCH // nb
    sdim = f_s.shape[1]
    tdim = f_t.shape[1]
    return pl.pallas_call(
        _proj_body,
        grid=(nb,),
        in_specs=[
            pl.BlockSpec((bb, sdim), lambda i: (i, 0)),
            pl.BlockSpec((bb, tdim), lambda i: (i, 0)),
            pl.BlockSpec((sdim, DIM), lambda i: (0, 0)),
            pl.BlockSpec((tdim, DIM), lambda i: (0, 0)),
            pl.BlockSpec((1, DIM), lambda i: (0, 0)),
            pl.BlockSpec((1, DIM), lambda i: (0, 0)),
        ],
        out_specs=[
            pl.BlockSpec((bb, DIM), lambda i: (i, 0)),
            pl.BlockSpec((bb, DIM), lambda i: (i, 0)),
        ],
        out_shape=[
            jax.ShapeDtypeStruct((BATCH, DIM), jnp.float32),
            jax.ShapeDtypeStruct((BATCH, DIM), jnp.float32),
        ],
    )(f_s, f_t, WsT, WtT, bs, bt)


SC_CHUNKS = (96, 96, 96, 96, 96, 48)
SC_OFFS = (0, 96, 192, 288, 384, 480)
DIMW = DIM // 2


def _sc_body(mi, mem1f, mem2f, cidx, idxp, v1h, v2h,
             sa_h, sb_h, pos1_h, pos2_h,
             cidx_cur, v1_blk, v2_blk, sa_blk, sb_blk,
             ra, rb,
             pidx_v, prow1_v, prow2_v,
             sem0, sem1, psem):
    c = lax.axis_index("c")
    s = lax.axis_index("s")
    w = s * 2 + c
    base_b = w * B_PER_W

    pltpu.sync_copy(v1h.at[pl.ds(base_b, B_PER_W)], v1_blk)
    pltpu.sync_copy(v2h.at[pl.ds(base_b, B_PER_W)], v2_blk)

    pltpu.sync_copy(idxp.at[pl.ds(base_b, B_PER_W)], pidx_v)
    pltpu.async_copy(mem1f.at[pidx_v], prow1_v, psem)
    pltpu.async_copy(mem2f.at[pidx_v], prow2_v, psem)

    lane = lax.iota(jnp.int32, LANES)
    rows = (ra, rb)
    sems = (sem0, sem1)

    def mk_copy(ci):
        p = ci % 2
        csz = SC_CHUNKS[ci]
        idxsl = cidx_cur.at[pl.ds(SC_OFFS[ci], csz)]
        return pltpu.make_async_copy(mi.at[idxsl],
                                     rows[p].at[pl.ds(0, csz)], sems[p])

    def issue(ci):
        mk_copy(ci).start()

    @pl.loop(0, B_PER_W)
    def _batch(bl):
        pltpu.sync_copy(cidx.at[base_b + bl], cidx_cur)
        issue(0)
        v1r = [plsc.bitcast(v1_blk.at[bl][pl.ds(LANES * j, LANES)],
                            jnp.bfloat16) for j in range(4)]
        v2r = [plsc.bitcast(v2_blk.at[bl][pl.ds(LANES * j, LANES)],
                            jnp.bfloat16) for j in range(4)]

        for ci, csz in enumerate(SC_CHUNKS):
            p = ci % 2
            mk_copy(ci).wait()
            if ci + 1 < len(SC_CHUNKS):
                issue(ci + 1)

            coff = SC_OFFS[ci]

            @pl.loop(0, csz // LANES)
            def _group(g):
                def bank(half, vr):
                    accs = []
                    for r in range(LANES):
                        row = g * LANES + r
                        acc = plsc.bitcast(
                            rows[p].at[row][pl.ds(half, LANES)],
                            jnp.bfloat16) * vr[0]
                        for j in range(1, 4):
                            acc = acc + plsc.bitcast(
                                rows[p].at[row][pl.ds(half + LANES * j, LANES)],
                                jnp.bfloat16) * vr[j]
                        accs.append(acc)
                    res = jnp.zeros((LANES,), jnp.float32)
                    for r in range(LANES):
                        lo, hi = plsc.unpack(accs[r],
                                             format=plsc.PackFormat.INTERLEAVED)
                        res = jnp.where(lane == r, jnp.sum(lo + hi), res)
                    return res

                base = pl.multiple_of(coff + g * LANES, LANES)
                sa_blk[bl, pl.ds(base, LANES)] = bank(DIMW, v1r)
                sb_blk[bl, pl.ds(base, LANES)] = bank(0, v2r)

    pltpu.sync_copy(sa_blk, sa_h.at[pl.ds(base_b, B_PER_W)])
    pltpu.sync_copy(sb_blk, sb_h.at[pl.ds(base_b, B_PER_W)])

    pltpu.make_async_copy(mem1f.at[pidx_v], prow1_v, psem).wait()
    pltpu.make_async_copy(mem2f.at[pidx_v], prow2_v, psem).wait()
    pltpu.sync_copy(prow1_v, pos1_h.at[pl.ds(base_b, B_PER_W)])
    pltpu.sync_copy(prow2_v, pos2_h.at[pl.ds(base_b, B_PER_W)])


def _sc_gather_dot(mi, mem1f, mem2f, cidx_p, idx, v1b, v2b):
    mesh = plsc.VectorSubcoreMesh(core_axis_name="c", subcore_axis_name="s")
    f32 = jnp.float32
    i32 = jnp.int32
    cp = pltpu.CompilerParams()
    if "needs_layout_passes" in pltpu.CompilerParams.__dataclass_fields__:
        cp = dataclasses.replace(cp, needs_layout_passes=False)
    rbuf = pltpu.VMEM((max(SC_CHUNKS), DIM), i32)
    kern = pl.kernel(
        _sc_body,
        out_type=[
            jax.ShapeDtypeStruct((BATCH, KP), f32),
            jax.ShapeDtypeStruct((BATCH, KP), f32),
            jax.ShapeDtypeStruct((BATCH, DIM), f32),
            jax.ShapeDtypeStruct((BATCH, DIM), f32),
        ],
        mesh=mesh,
        scratch_types=[
            pltpu.VMEM((KP,), i32),
            pltpu.VMEM((B_PER_W, DIMW), i32),
            pltpu.VMEM((B_PER_W, DIMW), i32),
            pltpu.VMEM((B_PER_W, KP), f32),
            pltpu.VMEM((B_PER_W, KP), f32),
            rbuf, rbuf,
            pltpu.VMEM((B_PER_W,), i32),
            pltpu.VMEM((B_PER_W, DIM), f32),
            pltpu.VMEM((B_PER_W, DIM), f32),
            pltpu.SemaphoreType.DMA,
            pltpu.SemaphoreType.DMA,
            pltpu.SemaphoreType.DMA,
        ],
        compiler_params=cp,
    )
    return kern(mi, mem1f, mem2f, cidx_p, idx, v1b, v2b)


def _tobf16_body(m1_ref, m2_ref, o_ref):
    o_ref[:, :DIM] = m1_ref[...].astype(jnp.bfloat16)
    o_ref[:, DIM:] = m2_ref[...].astype(jnp.bfloat16)


def _tobf16(mem1, mem2):
    rb = 2000
    nb = N_DATA // rb
    return pl.pallas_call(
        _tobf16_body,
        grid=(nb,),
        in_specs=[
            pl.BlockSpec((rb, DIM), lambda i: (i, 0)),
            pl.BlockSpec((rb, DIM), lambda i: (i, 0)),
        ],
        out_specs=pl.BlockSpec((rb, 2 * DIM), lambda i: (i, 0)),
        out_shape=jax.ShapeDtypeStruct((N_DATA, 2 * DIM), jnp.bfloat16),
    )(mem1, mem2)


def _loss_body(sa_ref, sb_ref, loss_ref):
    pn_m = float(NCE_K) / float(N_DATA)
    cden = pn_m + EPS
    kcol = lax.broadcasted_iota(jnp.int32, (BATCH, KP), 1)
    valid = kcol < K1
    negm = (kcol >= 1) & (kcol < K1)

    def one_side(s_ref):
        e = jnp.where(valid, jnp.exp(s_ref[...] * (1.0 / NCE_T)), 0.0)
        z = jnp.sum(e) * (float(N_DATA) / float(BATCH * K1))
        p = e / z
        pos = p[:, 0:1]
        pos_term = jnp.sum(jnp.log(pos / (pos + cden)))
        neg_term = jnp.sum(jnp.where(negm, jnp.log(pn_m / (p + cden)), 0.0))
        return -(pos_term + neg_term) / float(BATCH)

    loss_ref[0, 0] = one_side(sa_ref) + one_side(sb_ref)


def _loss(sa, sb):
    return pl.pallas_call(
        _loss_body,
        in_specs=[
            pl.BlockSpec((BATCH, KP), lambda: (0, 0)),
            pl.BlockSpec((BATCH, KP), lambda: (0, 0)),
        ],
        out_specs=pl.BlockSpec((1, 1), lambda: (0, 0), memory_space=pltpu.SMEM),
        out_shape=jax.ShapeDtypeStruct((1, 1), jnp.float32),
    )(sa, sb)


def _update_body(pos1_ref, pos2_ref, v1_ref, v2_ref, idxr_ref, idxc_ref,
                 r1_ref, r2_ref):
    l1 = pos1_ref[...] * MOMENTUM + v1_ref[...] * (1.0 - MOMENTUM)
    l1 = l1 / jnp.sqrt(jnp.sum(l1 * l1, axis=1, keepdims=True))
    l2 = pos2_ref[...] * MOMENTUM + v2_ref[...] * (1.0 - MOMENTUM)
    l2 = l2 / jnp.sqrt(jnp.sum(l2 * l2, axis=1, keepdims=True))

    iota_col = lax.broadcasted_iota(jnp.int32, (BATCH, BATCH), 1)
    same = idxc_ref[...] == idxr_ref[...]
    winner = jnp.max(jnp.where(same, iota_col, -1), axis=1, keepdims=True)
    onehot = (iota_col == winner).astype(jnp.float32)
    r1_ref[...] = jnp.dot(onehot, l1, preferred_element_type=jnp.float32)
    r2_ref[...] = jnp.dot(onehot, l2, preferred_element_type=jnp.float32)


def _update(pos1, pos2, v1, v2, idx):
    idxr = idx.reshape(1, BATCH)
    idxc = idx.reshape(BATCH, 1)
    return pl.pallas_call(
        _update_body,
        in_specs=[
            pl.BlockSpec((BATCH, DIM), lambda: (0, 0)),
            pl.BlockSpec((BATCH, DIM), lambda: (0, 0)),
            pl.BlockSpec((BATCH, DIM), lambda: (0, 0)),
            pl.BlockSpec((BATCH, DIM), lambda: (0, 0)),
            pl.BlockSpec((1, BATCH), lambda: (0, 0)),
            pl.BlockSpec((BATCH, 1), lambda: (0, 0)),
        ],
        out_specs=[
            pl.BlockSpec((BATCH, DIM), lambda: (0, 0)),
            pl.BlockSpec((BATCH, DIM), lambda: (0, 0)),
        ],
        out_shape=[
            jax.ShapeDtypeStruct((BATCH, DIM), jnp.float32),
            jax.ShapeDtypeStruct((BATCH, DIM), jnp.float32),
        ],
    )(pos1, pos2, v1, v2, idxr, idxc)


def _scatter_body(idx_ref, rows1_ref, rows2_ref, mem1_ref, mem2_ref,
                  out1_ref, out2_ref, sem1, sem2):
    def issue(i, _):
        r = idx_ref[i]
        pltpu.make_async_copy(rows1_ref.at[pl.ds(i, 1)], out1_ref.at[pl.ds(r, 1)],
                              sem1).start()
        pltpu.make_async_copy(rows2_ref.at[pl.ds(i, 1)], out2_ref.at[pl.ds(r, 1)],
                              sem2).start()
        return 0

    lax.fori_loop(0, BATCH, issue, 0)

    def drain(i, _):
        r = idx_ref[i]
        pltpu.make_async_copy(rows1_ref.at[pl.ds(i, 1)], out1_ref.at[pl.ds(r, 1)],
                              sem1).wait()
        pltpu.make_async_copy(rows2_ref.at[pl.ds(i, 1)], out2_ref.at[pl.ds(r, 1)],
                              sem2).wait()
        return 0

    lax.fori_loop(0, BATCH, drain, 0)


def _scatter(idx, rows1, rows2, mem1, mem2):
    return pl.pallas_call(
        _scatter_body,
        in_specs=[
            pl.BlockSpec(memory_space=pltpu.SMEM),
            pl.BlockSpec(memory_space=pltpu.VMEM),
            pl.BlockSpec(memory_space=pltpu.VMEM),
            pl.BlockSpec(memory_space=pl.ANY),
            pl.BlockSpec(memory_space=pl.ANY),
        ],
        out_specs=[
            pl.BlockSpec(memory_space=pl.ANY),
            pl.BlockSpec(memory_space=pl.ANY),
        ],
        out_shape=[
            jax.ShapeDtypeStruct((N_DATA, DIM), jnp.float32),
            jax.ShapeDtypeStruct((N_DATA, DIM), jnp.float32),
        ],
        input_output_aliases={3: 0, 4: 1},
        scratch_shapes=[pltpu.SemaphoreType.DMA, pltpu.SemaphoreType.DMA],
    )(idx, rows1, rows2, mem1, mem2)


def kernel(f_s, f_t, idx, contrast_idx, Ws, bs, Wt, bt, memory_v1, memory_v2):
    idx = idx.astype(jnp.int32)
    cidx = contrast_idx.astype(jnp.int32)
    cidx_p = jnp.concatenate(
        [cidx, jnp.broadcast_to(cidx[:, :1], (BATCH, KP - K1))], axis=1)

    v1, v2 = _project(f_s, f_t, Ws.T, Wt.T,
                      bs.reshape(1, DIM), bt.reshape(1, DIM))

    mb = _tobf16(memory_v1, memory_v2)
    mi = lax.bitcast_convert_type(mb.reshape(N_DATA, DIM, 2), jnp.int32)

    v1i = lax.bitcast_convert_type(
        v1.astype(jnp.bfloat16).reshape(BATCH, DIMW, 2), jnp.int32)
    v2i = lax.bitcast_convert_type(
        v2.astype(jnp.bfloat16).reshape(BATCH, DIMW, 2), jnp.int32)
    sa, sb, pos1, pos2 = _sc_gather_dot(mi, memory_v1, memory_v2,
                                        cidx_p, idx, v1i, v2i)

    loss = _loss(sa, sb).reshape(1)

    rows1, rows2 = _update(pos1, pos2, v1, v2, idx)

    new_mem1, new_mem2 = _scatter(idx, rows1, rows2, memory_v1, memory_v2)

    return loss, new_mem1, new_mem2
